# Initial kernel scaffold; baseline (speedup 1.0000x reference)
import functools

import jax
import jax.numpy as jnp
from jax.experimental import pallas as pl
from jax.experimental.pallas import tpu as pltpu

TOPK = 2
ALPHA = 1.702
LIMIT = 7.0


def _router_block(x_blk, rw, rb):
    E = rw.shape[0]
    logits = jax.lax.dot_general(
        x_blk, rw, (((1,), (1,)), ((), ())),
        preferred_element_type=jnp.float32,
        precision=jax.lax.Precision.HIGHEST,
    ) + rb
    iota = jax.lax.broadcasted_iota(jnp.int32, logits.shape, 1)
    m0 = jnp.max(logits, axis=1, keepdims=True)
    a0 = jnp.min(jnp.where(logits == m0, iota, E), axis=1, keepdims=True)
    l2 = jnp.where(iota == a0, -jnp.inf, logits)
    m1 = jnp.max(l2, axis=1, keepdims=True)
    a1 = jnp.min(jnp.where(l2 == m1, iota, E), axis=1, keepdims=True)
    e1 = jnp.exp(m1 - m0)
    s = 1.0 + e1
    w0 = 1.0 / s
    w1 = e1 / s
    scores = (jnp.where(iota == a0, w0, 0.0) + jnp.where(iota == a1, w1, 0.0))
    return scores, w0, w1, a0, a1


def _dense_kernel(x_ref, rw_ref, rb_ref, gw_ref, gb_ref, uw_ref, ub_ref,
                  dw_ref, db_ref, out_ref, scores_ref, acc_ref):
    e = pl.program_id(0)
    x = x_ref[...]
    scores, _, _, _, _ = _router_block(x, rw_ref[...], rb_ref[...])
    scores_ref[...] = scores

    iota = jax.lax.broadcasted_iota(jnp.int32, scores.shape, 1)
    g = jnp.sum(jnp.where(iota == e, scores, 0.0), axis=1, keepdims=True)

    gate = jax.lax.dot_general(
        x, gw_ref[0], (((1,), (1,)), ((), ())),
        preferred_element_type=jnp.float32,
        precision=jax.lax.Precision.HIGHEST) + gb_ref[...]
    up = jax.lax.dot_general(
        x, uw_ref[0], (((1,), (1,)), ((), ())),
        preferred_element_type=jnp.float32,
        precision=jax.lax.Precision.HIGHEST) + ub_ref[...]
    gate = jnp.minimum(gate, LIMIT)
    up = jnp.clip(up, -LIMIT, LIMIT)
    glu = gate * (1.0 / (1.0 + jnp.exp(-ALPHA * gate)))
    act = (up + 1.0) * glu
    down = jax.lax.dot_general(
        act, dw_ref[0], (((1,), (1,)), ((), ())),
        preferred_element_type=jnp.float32,
        precision=jax.lax.Precision.HIGHEST) + db_ref[...]
    contrib = g * down

    t = pl.program_id(1)
    tb = x.shape[0]

    @pl.when(e == 0)
    def _():
        acc_ref[pl.ds(t * tb, tb), :] = contrib

    @pl.when(e > 0)
    def _():
        acc_ref[pl.ds(t * tb, tb), :] += contrib

    out_ref[...] = acc_ref[pl.ds(t * tb, tb), :]


@jax.jit
def kernel(hidden_states, router_w, router_b, gate_w, gate_b, up_w, up_b,
           down_w, down_b):
    B, T, H = hidden_states.shape
    x = hidden_states.reshape(T, H)
    E, I, _ = gate_w.shape
    TB = 256
    n_t = T // TB

    out, scores = pl.pallas_call(
        _dense_kernel,
        grid=(E, n_t),
        in_specs=[
            pl.BlockSpec((TB, H), lambda e, t: (t, 0)),
            pl.BlockSpec((E, H), lambda e, t: (0, 0)),
            pl.BlockSpec((1, E), lambda e, t: (0, 0)),
            pl.BlockSpec((1, I, H), lambda e, t: (e, 0, 0)),
            pl.BlockSpec((1, I), lambda e, t: (e, 0)),
            pl.BlockSpec((1, I, H), lambda e, t: (e, 0, 0)),
            pl.BlockSpec((1, I), lambda e, t: (e, 0)),
            pl.BlockSpec((1, H, I), lambda e, t: (e, 0, 0)),
            pl.BlockSpec((1, H), lambda e, t: (e, 0)),
        ],
        out_specs=[
            pl.BlockSpec((TB, H), lambda e, t: (t, 0)),
            pl.BlockSpec((TB, E), lambda e, t: (t, 0)),
        ],
        out_shape=[
            jax.ShapeDtypeStruct((T, H), jnp.float32),
            jax.ShapeDtypeStruct((T, E), jnp.float32),
        ],
        scratch_shapes=[pltpu.VMEM((T, H), jnp.float32)],
        compiler_params=pltpu.CompilerParams(
            dimension_semantics=("arbitrary", "arbitrary")),
    )(x, router_w, router_b.reshape(1, E), gate_w, gate_b, up_w, up_b,
      down_w, down_b)

    return (out.reshape(B, T, H), scores)


# speedup vs baseline: 1.5267x; 1.5267x over previous
import functools

import jax
import jax.numpy as jnp
from jax.experimental import pallas as pl
from jax.experimental.pallas import tpu as pltpu

TOPK = 2
ALPHA = 1.702
LIMIT = 7.0


def _bdot(a, b):
    return jax.lax.dot_general(
        a.astype(jnp.bfloat16), b.astype(jnp.bfloat16),
        (((1,), (1,)), ((), ())),
        preferred_element_type=jnp.float32)

def _router_block(x_blk, rw, rb):
    E = rw.shape[0]
    logits = _bdot(x_blk, rw) + rb
    iota = jax.lax.broadcasted_iota(jnp.int32, logits.shape, 1)
    m0 = jnp.max(logits, axis=1, keepdims=True)
    a0 = jnp.min(jnp.where(logits == m0, iota, E), axis=1, keepdims=True)
    l2 = jnp.where(iota == a0, -jnp.inf, logits)
    m1 = jnp.max(l2, axis=1, keepdims=True)
    a1 = jnp.min(jnp.where(l2 == m1, iota, E), axis=1, keepdims=True)
    e1 = jnp.exp(m1 - m0)
    s = 1.0 + e1
    w0 = 1.0 / s
    w1 = e1 / s
    scores = (jnp.where(iota == a0, w0, 0.0) + jnp.where(iota == a1, w1, 0.0))
    return scores, w0, w1, a0, a1


def _dense_kernel(x_ref, rw_ref, rb_ref, gw_ref, gb_ref, uw_ref, ub_ref,
                  dw_ref, db_ref, out_ref, scores_ref, acc_ref):
    e = pl.program_id(0)
    x = x_ref[...]
    scores, _, _, _, _ = _router_block(x, rw_ref[...], rb_ref[...])
    scores_ref[...] = scores

    iota = jax.lax.broadcasted_iota(jnp.int32, scores.shape, 1)
    g = jnp.sum(jnp.where(iota == e, scores, 0.0), axis=1, keepdims=True)

    gate = _bdot(x, gw_ref[0]) + gb_ref[0]
    up = _bdot(x, uw_ref[0]) + ub_ref[0]
    gate = jnp.minimum(gate, LIMIT)
    up = jnp.clip(up, -LIMIT, LIMIT)
    glu = gate * (1.0 / (1.0 + jnp.exp(-ALPHA * gate)))
    act = (up + 1.0) * glu
    down = _bdot(act, dw_ref[0]) + db_ref[0]
    contrib = g * down

    t = pl.program_id(1)
    tb = x.shape[0]

    @pl.when(e == 0)
    def _():
        acc_ref[pl.ds(t * tb, tb), :] = contrib

    @pl.when(e > 0)
    def _():
        acc_ref[pl.ds(t * tb, tb), :] += contrib

    out_ref[...] = acc_ref[pl.ds(t * tb, tb), :]


@jax.jit
def kernel(hidden_states, router_w, router_b, gate_w, gate_b, up_w, up_b,
           down_w, down_b):
    B, T, H = hidden_states.shape
    x = hidden_states.reshape(T, H)
    E, I, _ = gate_w.shape
    TB = 256
    n_t = T // TB

    out, scores = pl.pallas_call(
        _dense_kernel,
        grid=(E, n_t),
        in_specs=[
            pl.BlockSpec((TB, H), lambda e, t: (t, 0)),
            pl.BlockSpec((E, H), lambda e, t: (0, 0)),
            pl.BlockSpec((1, E), lambda e, t: (0, 0)),
            pl.BlockSpec((1, I, H), lambda e, t: (e, 0, 0)),
            pl.BlockSpec((1, 1, I), lambda e, t: (e, 0, 0)),
            pl.BlockSpec((1, I, H), lambda e, t: (e, 0, 0)),
            pl.BlockSpec((1, 1, I), lambda e, t: (e, 0, 0)),
            pl.BlockSpec((1, H, I), lambda e, t: (e, 0, 0)),
            pl.BlockSpec((1, 1, H), lambda e, t: (e, 0, 0)),
        ],
        out_specs=[
            pl.BlockSpec((TB, H), lambda e, t: (t, 0)),
            pl.BlockSpec((TB, E), lambda e, t: (t, 0)),
        ],
        out_shape=[
            jax.ShapeDtypeStruct((T, H), jnp.float32),
            jax.ShapeDtypeStruct((T, E), jnp.float32),
        ],
        scratch_shapes=[pltpu.VMEM((T, H), jnp.float32)],
        compiler_params=pltpu.CompilerParams(
            dimension_semantics=("arbitrary", "arbitrary")),
    )(x, router_w, router_b.reshape(1, E), gate_w, gate_b.reshape(E, 1, I),
      up_w, up_b.reshape(E, 1, I), down_w, down_b.reshape(E, 1, H))

    return (out.reshape(B, T, H), scores)


# speedup vs baseline: 2.0138x; 1.3190x over previous
import functools

import jax
import jax.numpy as jnp
from jax import lax
from jax.experimental import pallas as pl
from jax.experimental.pallas import tpu as pltpu
from jax.experimental.pallas import tpu_sc as plsc

TOPK = 2
ALPHA = 1.702
LIMIT = 7.0

T = 2048
H = 1024
I = 1024
E = 8
BLK = 256
A = T * TOPK
PMAX = A + E * BLK
NBMAX = A // BLK + E
NW = 32
TPW = T // NW


def _bdot(a, b):
    return jax.lax.dot_general(
        a.astype(jnp.bfloat16), b.astype(jnp.bfloat16),
        (((1,), (1,)), ((), ())),
        preferred_element_type=jnp.float32)


def _incl_scan_rows(m):
    d = 1
    n = m.shape[0]
    while d < n:
        z = jnp.zeros((d, m.shape[1]), m.dtype)
        m = m + jnp.concatenate([z, m[:-d, :]], axis=0)
        d *= 2
    return m


def _incl_scan_lanes(v):
    d = 1
    n = v.shape[1]
    while d < n:
        z = jnp.zeros((v.shape[0], d), v.dtype)
        v = v + jnp.concatenate([z, v[:, :-d]], axis=1)
        d *= 2
    return v


def _plan_kernel(x_ref, rw_ref, rb_ref, scores_ref, d0_ref, d1_ref,
                 w0_ref, w1_ref, be_ref, na_ref):
    x = x_ref[...]
    logits = _bdot(x, rw_ref[...]) + rb_ref[...]
    iota = lax.broadcasted_iota(jnp.int32, logits.shape, 1)
    m0 = jnp.max(logits, axis=1, keepdims=True)
    a0 = jnp.min(jnp.where(logits == m0, iota, E), axis=1, keepdims=True)
    l2 = jnp.where(iota == a0, -jnp.inf, logits)
    m1 = jnp.max(l2, axis=1, keepdims=True)
    a1 = jnp.min(jnp.where(l2 == m1, iota, E), axis=1, keepdims=True)
    e1 = jnp.exp(m1 - m0)
    s = 1.0 + e1
    w0 = 1.0 / s
    w1 = e1 / s
    oh0 = (iota == a0)
    oh1 = (iota == a1)
    scores_ref[...] = (jnp.where(oh0, w0, 0.0) + jnp.where(oh1, w1, 0.0))
    w0_ref[...] = w0
    w1_ref[...] = w1

    oh0_i = oh0.astype(jnp.int32)
    oh1_i = oh1.astype(jnp.int32)
    csum0 = _incl_scan_rows(oh0_i)
    csum1 = _incl_scan_rows(oh1_i)
    counts0 = csum0[T - 1:T, :]
    counts1 = csum1[T - 1:T, :]
    counts = counts0 + counts1
    padded = ((counts + (BLK - 1)) // BLK) * BLK
    nblk = padded // BLK
    end_blk = _incl_scan_lanes(nblk)
    offsets = (end_blk - nblk) * BLK
    na_ref[...] = end_blk[:, E - 1:E]

    rank0 = jnp.sum(oh0_i * csum0, axis=1, keepdims=True) - 1
    rank1 = jnp.sum(oh1_i * csum1, axis=1, keepdims=True) - 1
    offs0 = jnp.sum(oh0_i * offsets, axis=1, keepdims=True)
    offs1 = jnp.sum(oh1_i * offsets, axis=1, keepdims=True)
    c0sel = jnp.sum(oh1_i * counts0, axis=1, keepdims=True)
    d0_ref[...] = offs0 + rank0
    d1_ref[...] = offs1 + c0sel + rank1

    iota_b = lax.broadcasted_iota(jnp.int32, (NBMAX, E), 0)
    be = jnp.sum((end_blk <= iota_b).astype(jnp.int32), axis=1, keepdims=True)
    be_ref[...] = jnp.minimum(be, E - 1)


def _plan(x, router_w, router_b):
    return pl.pallas_call(
        _plan_kernel,
        out_shape=[
            jax.ShapeDtypeStruct((T, E), jnp.float32),
            jax.ShapeDtypeStruct((T, 1), jnp.int32),
            jax.ShapeDtypeStruct((T, 1), jnp.int32),
            jax.ShapeDtypeStruct((T, 1), jnp.float32),
            jax.ShapeDtypeStruct((T, 1), jnp.float32),
            jax.ShapeDtypeStruct((NBMAX, 1), jnp.int32),
            jax.ShapeDtypeStruct((1, 1), jnp.int32),
        ],
    )(x, router_w, router_b.reshape(1, E))


def _dispatch_body(x_hbm, d0_hbm, d1_hbm, xs_hbm,
                   xv, i0v, i1v, sem0, sem1):
    wid = lax.axis_index("s") * 2 + lax.axis_index("c")
    base = wid * TPW
    pltpu.sync_copy(x_hbm.at[pl.ds(base, TPW)], xv)
    pltpu.sync_copy(d0_hbm.at[pl.ds(base, TPW)], i0v)
    pltpu.sync_copy(d1_hbm.at[pl.ds(base, TPW)], i1v)
    c0 = pltpu.async_copy(xv, xs_hbm.at[i0v], sem0)
    c1 = pltpu.async_copy(xv, xs_hbm.at[i1v], sem1)
    c0.wait()
    c1.wait()


def _dispatch(x, d0f, d1f):
    k = functools.partial(
        pl.kernel,
        out_type=jax.ShapeDtypeStruct((PMAX, H), jnp.float32),
        mesh=plsc.VectorSubcoreMesh(core_axis_name="c", subcore_axis_name="s"),
        scratch_types=[
            pltpu.VMEM((TPW, H), jnp.float32),
            pltpu.VMEM((TPW,), jnp.int32),
            pltpu.VMEM((TPW,), jnp.int32),
            pltpu.SemaphoreType.DMA,
            pltpu.SemaphoreType.DMA,
        ],
    )(_dispatch_body)
    return k(x, d0f, d1f)


def _gmlp_kernel(be_ref, na_ref, xs_ref, gw_ref, gb_ref, uw_ref, ub_ref,
                 dw_ref, db_ref, y_ref):
    b = pl.program_id(0)

    @pl.when(b < na_ref[0])
    def _():
        x = xs_ref[...]
        gate = _bdot(x, gw_ref[0]) + gb_ref[0]
        up = _bdot(x, uw_ref[0]) + ub_ref[0]
        gate = jnp.minimum(gate, LIMIT)
        up = jnp.clip(up, -LIMIT, LIMIT)
        glu = gate * (1.0 / (1.0 + jnp.exp(-ALPHA * gate)))
        act = (up + 1.0) * glu
        y_ref[...] = _bdot(act, dw_ref[0]) + db_ref[0]


def _gmlp(be, na, xs, gate_w, gate_b, up_w, up_b, down_w, down_b):
    def _bc(b, be_r, na_r):
        return jnp.minimum(b, na_r[0] - 1)

    grid_spec = pltpu.PrefetchScalarGridSpec(
        num_scalar_prefetch=2,
        grid=(NBMAX,),
        in_specs=[
            pl.BlockSpec((BLK, H), lambda b, be_r, na_r: (_bc(b, be_r, na_r), 0)),
            pl.BlockSpec((1, I, H),
                         lambda b, be_r, na_r: (be_r[_bc(b, be_r, na_r)], 0, 0)),
            pl.BlockSpec((1, 1, I),
                         lambda b, be_r, na_r: (be_r[_bc(b, be_r, na_r)], 0, 0)),
            pl.BlockSpec((1, I, H),
                         lambda b, be_r, na_r: (be_r[_bc(b, be_r, na_r)], 0, 0)),
            pl.BlockSpec((1, 1, I),
                         lambda b, be_r, na_r: (be_r[_bc(b, be_r, na_r)], 0, 0)),
            pl.BlockSpec((1, H, I),
                         lambda b, be_r, na_r: (be_r[_bc(b, be_r, na_r)], 0, 0)),
            pl.BlockSpec((1, 1, H),
                         lambda b, be_r, na_r: (be_r[_bc(b, be_r, na_r)], 0, 0)),
        ],
        out_specs=pl.BlockSpec((BLK, H),
                               lambda b, be_r, na_r: (_bc(b, be_r, na_r), 0)),
    )
    return pl.pallas_call(
        _gmlp_kernel,
        grid_spec=grid_spec,
        out_shape=jax.ShapeDtypeStruct((PMAX, H), jnp.float32),
        compiler_params=pltpu.CompilerParams(
            dimension_semantics=("arbitrary",)),
    )(be, na, xs, gate_w, gate_b.reshape(E, 1, I), up_w,
      up_b.reshape(E, 1, I), down_w, down_b.reshape(E, 1, H))


def _combine_body(y_hbm, d0_hbm, d1_hbm, y0_hbm, y1_hbm,
                  g0, g1, j0, j1, sem0, sem1):
    wid = lax.axis_index("s") * 2 + lax.axis_index("c")
    CH = 32
    for c in range(TPW // CH):
        tbase = wid * TPW + c * CH
        pltpu.sync_copy(d0_hbm.at[pl.ds(tbase, CH)], j0)
        pltpu.sync_copy(d1_hbm.at[pl.ds(tbase, CH)], j1)
        c0 = pltpu.async_copy(y_hbm.at[j0], g0, sem0)
        c1 = pltpu.async_copy(y_hbm.at[j1], g1, sem1)
        c0.wait()
        c1.wait()
        pltpu.sync_copy(g0, y0_hbm.at[pl.ds(tbase, CH)])
        pltpu.sync_copy(g1, y1_hbm.at[pl.ds(tbase, CH)])


def _combine(y, d0f, d1f):
    k = functools.partial(
        pl.kernel,
        out_type=[
            jax.ShapeDtypeStruct((T, H), jnp.float32),
            jax.ShapeDtypeStruct((T, H), jnp.float32),
        ],
        mesh=plsc.VectorSubcoreMesh(core_axis_name="c", subcore_axis_name="s"),
        scratch_types=[
            pltpu.VMEM((32, H), jnp.float32),
            pltpu.VMEM((32, H), jnp.float32),
            pltpu.VMEM((32,), jnp.int32),
            pltpu.VMEM((32,), jnp.int32),
            pltpu.SemaphoreType.DMA,
            pltpu.SemaphoreType.DMA,
        ],
    )(_combine_body)
    return k(y, d0f, d1f)


def _add_kernel(a_ref, b_ref, wa_ref, wb_ref, o_ref):
    o_ref[...] = wa_ref[...] * a_ref[...] + wb_ref[...] * b_ref[...]


def _add(a, b, wa, wb):
    return pl.pallas_call(
        _add_kernel,
        grid=(8,),
        in_specs=[
            pl.BlockSpec((T // 8, H), lambda i: (i, 0)),
            pl.BlockSpec((T // 8, H), lambda i: (i, 0)),
            pl.BlockSpec((T // 8, 1), lambda i: (i, 0)),
            pl.BlockSpec((T // 8, 1), lambda i: (i, 0)),
        ],
        out_specs=pl.BlockSpec((T // 8, H), lambda i: (i, 0)),
        out_shape=jax.ShapeDtypeStruct((T, H), jnp.float32),
    )(a, b, wa, wb)


@jax.jit
def kernel(hidden_states, router_w, router_b, gate_w, gate_b, up_w, up_b,
           down_w, down_b):
    B, Tq, Hq = hidden_states.shape
    x = hidden_states.reshape(T, H)

    scores, d0, d1, w0, w1, be, na = _plan(x, router_w, router_b)
    d0f = d0.reshape(T)
    d1f = d1.reshape(T)

    xs = _dispatch(x, d0f, d1f)
    y = _gmlp(be.reshape(NBMAX), na.reshape(1), xs,
              gate_w, gate_b, up_w, up_b, down_w, down_b)
    y0, y1 = _combine(y, d0f, d1f)
    out = _add(y0, y1, w0, w1)

    return (out.reshape(B, Tq, Hq), scores)


# speedup vs baseline: 2.0208x; 1.0035x over previous
import functools

import jax
import jax.numpy as jnp
from jax import lax
from jax.experimental import pallas as pl
from jax.experimental.pallas import tpu as pltpu
from jax.experimental.pallas import tpu_sc as plsc

TOPK = 2
ALPHA = 1.702
LIMIT = 7.0

T = 2048
H = 1024
I = 1024
E = 8
BLK = 256
A = T * TOPK
PMAX = A + E * BLK
NBMAX = A // BLK + E
NW = 32
TPW = T // NW


def _bdot(a, b):
    return jax.lax.dot_general(
        a, b, (((1,), (1,)), ((), ())),
        preferred_element_type=jnp.float32)


def _incl_scan_rows(m):
    d = 1
    n = m.shape[0]
    while d < n:
        z = jnp.zeros((d, m.shape[1]), m.dtype)
        m = m + jnp.concatenate([z, m[:-d, :]], axis=0)
        d *= 2
    return m


def _incl_scan_lanes(v):
    d = 1
    n = v.shape[1]
    while d < n:
        z = jnp.zeros((v.shape[0], d), v.dtype)
        v = v + jnp.concatenate([z, v[:, :-d]], axis=1)
        d *= 2
    return v


def _plan_kernel(x_ref, rw_ref, rb_ref, scores_ref, d0_ref, d1_ref,
                 w0_ref, w1_ref, be_ref, na_ref):
    x = x_ref[...]
    logits = _bdot(x, rw_ref[...]) + rb_ref[...]
    iota = lax.broadcasted_iota(jnp.int32, logits.shape, 1)
    m0 = jnp.max(logits, axis=1, keepdims=True)
    a0 = jnp.min(jnp.where(logits == m0, iota, E), axis=1, keepdims=True)
    l2 = jnp.where(iota == a0, -jnp.inf, logits)
    m1 = jnp.max(l2, axis=1, keepdims=True)
    a1 = jnp.min(jnp.where(l2 == m1, iota, E), axis=1, keepdims=True)
    e1 = jnp.exp(m1 - m0)
    s = 1.0 + e1
    w0 = 1.0 / s
    w1 = e1 / s
    oh0 = (iota == a0)
    oh1 = (iota == a1)
    scores_ref[...] = (jnp.where(oh0, w0, 0.0) + jnp.where(oh1, w1, 0.0))
    w0_ref[...] = w0
    w1_ref[...] = w1

    oh0_i = oh0.astype(jnp.int32)
    oh1_i = oh1.astype(jnp.int32)
    csum0 = _incl_scan_rows(oh0_i)
    csum1 = _incl_scan_rows(oh1_i)
    counts0 = csum0[T - 1:T, :]
    counts1 = csum1[T - 1:T, :]
    counts = counts0 + counts1
    padded = ((counts + (BLK - 1)) // BLK) * BLK
    nblk = padded // BLK
    end_blk = _incl_scan_lanes(nblk)
    offsets = (end_blk - nblk) * BLK
    na_ref[...] = end_blk[:, E - 1:E]

    rank0 = jnp.sum(oh0_i * csum0, axis=1, keepdims=True) - 1
    rank1 = jnp.sum(oh1_i * csum1, axis=1, keepdims=True) - 1
    offs0 = jnp.sum(oh0_i * offsets, axis=1, keepdims=True)
    offs1 = jnp.sum(oh1_i * offsets, axis=1, keepdims=True)
    c0sel = jnp.sum(oh1_i * counts0, axis=1, keepdims=True)
    d0_ref[...] = offs0 + rank0
    d1_ref[...] = offs1 + c0sel + rank1

    iota_b = lax.broadcasted_iota(jnp.int32, (NBMAX, E), 0)
    be = jnp.sum((end_blk <= iota_b).astype(jnp.int32), axis=1, keepdims=True)
    be_ref[...] = jnp.minimum(be, E - 1)


def _plan(x, router_w, router_b):
    return pl.pallas_call(
        _plan_kernel,
        out_shape=[
            jax.ShapeDtypeStruct((T, E), jnp.float32),
            jax.ShapeDtypeStruct((T, 1), jnp.int32),
            jax.ShapeDtypeStruct((T, 1), jnp.int32),
            jax.ShapeDtypeStruct((T, 1), jnp.float32),
            jax.ShapeDtypeStruct((T, 1), jnp.float32),
            jax.ShapeDtypeStruct((NBMAX, 1), jnp.int32),
            jax.ShapeDtypeStruct((1, 1), jnp.int32),
        ],
    )(x, router_w, router_b.reshape(1, E))


def _dispatch_body(x_hbm, d0_hbm, d1_hbm, xs_hbm,
                   xv, i0v, i1v, sem0, sem1):
    wid = lax.axis_index("s") * 2 + lax.axis_index("c")
    base = wid * TPW
    pltpu.sync_copy(x_hbm.at[pl.ds(base, TPW)], xv)
    pltpu.sync_copy(d0_hbm.at[pl.ds(base, TPW)], i0v)
    pltpu.sync_copy(d1_hbm.at[pl.ds(base, TPW)], i1v)
    c0 = pltpu.async_copy(xv, xs_hbm.at[i0v], sem0)
    c1 = pltpu.async_copy(xv, xs_hbm.at[i1v], sem1)
    c0.wait()
    c1.wait()


def _dispatch(x, d0f, d1f):
    k = functools.partial(
        pl.kernel,
        out_type=jax.ShapeDtypeStruct((PMAX, H), jnp.float32),
        mesh=plsc.VectorSubcoreMesh(core_axis_name="c", subcore_axis_name="s"),
        scratch_types=[
            pltpu.VMEM((TPW, H), jnp.float32),
            pltpu.VMEM((TPW,), jnp.int32),
            pltpu.VMEM((TPW,), jnp.int32),
            pltpu.SemaphoreType.DMA,
            pltpu.SemaphoreType.DMA,
        ],
    )(_dispatch_body)
    return k(x, d0f, d1f)


def _gmlp_kernel(be_ref, na_ref, xs_ref, gw_ref, gb_ref, uw_ref, ub_ref,
                 dw_ref, db_ref, y_ref):
    b = pl.program_id(0)

    @pl.when(b < na_ref[0])
    def _():
        x = xs_ref[...]
        gate = _bdot(x, gw_ref[0]) + gb_ref[0]
        up = _bdot(x, uw_ref[0]) + ub_ref[0]
        gate = jnp.minimum(gate, LIMIT)
        up = jnp.clip(up, -LIMIT, LIMIT)
        glu = gate * (1.0 / (1.0 + jnp.exp(-ALPHA * gate)))
        act = (up + 1.0) * glu
        y_ref[...] = _bdot(act, dw_ref[0]) + db_ref[0]


def _gmlp(be, na, xs, gate_w, gate_b, up_w, up_b, down_w, down_b):
    def _bc(b, be_r, na_r):
        return jnp.minimum(b, na_r[0] - 1)

    grid_spec = pltpu.PrefetchScalarGridSpec(
        num_scalar_prefetch=2,
        grid=(NBMAX,),
        in_specs=[
            pl.BlockSpec((BLK, H), lambda b, be_r, na_r: (_bc(b, be_r, na_r), 0)),
            pl.BlockSpec((1, I, H),
                         lambda b, be_r, na_r: (be_r[_bc(b, be_r, na_r)], 0, 0)),
            pl.BlockSpec((1, 1, I),
                         lambda b, be_r, na_r: (be_r[_bc(b, be_r, na_r)], 0, 0)),
            pl.BlockSpec((1, I, H),
                         lambda b, be_r, na_r: (be_r[_bc(b, be_r, na_r)], 0, 0)),
            pl.BlockSpec((1, 1, I),
                         lambda b, be_r, na_r: (be_r[_bc(b, be_r, na_r)], 0, 0)),
            pl.BlockSpec((1, H, I),
                         lambda b, be_r, na_r: (be_r[_bc(b, be_r, na_r)], 0, 0)),
            pl.BlockSpec((1, 1, H),
                         lambda b, be_r, na_r: (be_r[_bc(b, be_r, na_r)], 0, 0)),
        ],
        out_specs=pl.BlockSpec((BLK, H),
                               lambda b, be_r, na_r: (_bc(b, be_r, na_r), 0)),
    )
    return pl.pallas_call(
        _gmlp_kernel,
        grid_spec=grid_spec,
        out_shape=jax.ShapeDtypeStruct((PMAX, H), jnp.float32),
        compiler_params=pltpu.CompilerParams(
            dimension_semantics=("arbitrary",)),
    )(be, na, xs, gate_w, gate_b.reshape(E, 1, I), up_w,
      up_b.reshape(E, 1, I), down_w, down_b.reshape(E, 1, H))


def _combine_body(y_hbm, d0_hbm, d1_hbm, y0_hbm, y1_hbm,
                  g0, g1, j0, j1, sem0, sem1):
    wid = lax.axis_index("s") * 2 + lax.axis_index("c")
    CH = 32
    for c in range(TPW // CH):
        tbase = wid * TPW + c * CH
        pltpu.sync_copy(d0_hbm.at[pl.ds(tbase, CH)], j0)
        pltpu.sync_copy(d1_hbm.at[pl.ds(tbase, CH)], j1)
        c0 = pltpu.async_copy(y_hbm.at[j0], g0, sem0)
        c1 = pltpu.async_copy(y_hbm.at[j1], g1, sem1)
        c0.wait()
        c1.wait()
        pltpu.sync_copy(g0, y0_hbm.at[pl.ds(tbase, CH)])
        pltpu.sync_copy(g1, y1_hbm.at[pl.ds(tbase, CH)])


def _combine(y, d0f, d1f):
    k = functools.partial(
        pl.kernel,
        out_type=[
            jax.ShapeDtypeStruct((T, H), jnp.float32),
            jax.ShapeDtypeStruct((T, H), jnp.float32),
        ],
        mesh=plsc.VectorSubcoreMesh(core_axis_name="c", subcore_axis_name="s"),
        scratch_types=[
            pltpu.VMEM((32, H), jnp.float32),
            pltpu.VMEM((32, H), jnp.float32),
            pltpu.VMEM((32,), jnp.int32),
            pltpu.VMEM((32,), jnp.int32),
            pltpu.SemaphoreType.DMA,
            pltpu.SemaphoreType.DMA,
        ],
    )(_combine_body)
    return k(y, d0f, d1f)


def _add_kernel(a_ref, b_ref, wa_ref, wb_ref, o_ref):
    o_ref[...] = wa_ref[...] * a_ref[...] + wb_ref[...] * b_ref[...]


def _add(a, b, wa, wb):
    return pl.pallas_call(
        _add_kernel,
        grid=(8,),
        in_specs=[
            pl.BlockSpec((T // 8, H), lambda i: (i, 0)),
            pl.BlockSpec((T // 8, H), lambda i: (i, 0)),
            pl.BlockSpec((T // 8, 1), lambda i: (i, 0)),
            pl.BlockSpec((T // 8, 1), lambda i: (i, 0)),
        ],
        out_specs=pl.BlockSpec((T // 8, H), lambda i: (i, 0)),
        out_shape=jax.ShapeDtypeStruct((T, H), jnp.float32),
    )(a, b, wa, wb)


@jax.jit
def kernel(hidden_states, router_w, router_b, gate_w, gate_b, up_w, up_b,
           down_w, down_b):
    B, Tq, Hq = hidden_states.shape
    x = hidden_states.reshape(T, H)

    scores, d0, d1, w0, w1, be, na = _plan(x, router_w, router_b)
    d0f = d0.reshape(T)
    d1f = d1.reshape(T)

    xs = _dispatch(x, d0f, d1f)
    y = _gmlp(be.reshape(NBMAX), na.reshape(1), xs,
              gate_w, gate_b, up_w, up_b, down_w, down_b)
    y0, y1 = _combine(y, d0f, d1f)
    out = _add(y0, y1, w0, w1)

    return (out.reshape(B, Tq, Hq), scores)


# speedup vs baseline: 2.0652x; 1.0219x over previous
import functools

import jax
import jax.numpy as jnp
from jax import lax
from jax.experimental import pallas as pl
from jax.experimental.pallas import tpu as pltpu
from jax.experimental.pallas import tpu_sc as plsc

TOPK = 2
ALPHA = 1.702
LIMIT = 7.0

T = 2048
H = 1024
I = 1024
E = 8
BLK = 256
A = T * TOPK
PMAX = A + E * BLK
NBMAX = A // BLK + E
NW = 32
TPW = T // NW


def _bdot(a, b):
    return jax.lax.dot_general(
        a, b, (((1,), (1,)), ((), ())),
        preferred_element_type=jnp.float32)


def _incl_scan_rows(m):
    d = 1
    n = m.shape[0]
    while d < n:
        z = jnp.zeros((d, m.shape[1]), m.dtype)
        m = m + jnp.concatenate([z, m[:-d, :]], axis=0)
        d *= 2
    return m


def _incl_scan_lanes(v):
    d = 1
    n = v.shape[1]
    while d < n:
        z = jnp.zeros((v.shape[0], d), v.dtype)
        v = v + jnp.concatenate([z, v[:, :-d]], axis=1)
        d *= 2
    return v


def _plan_kernel(x_ref, rw_ref, rb_ref, scores_ref, d0_ref, d1_ref,
                 w0_ref, w1_ref, be_ref, na_ref):
    x = x_ref[...]
    logits = _bdot(x, rw_ref[...]) + rb_ref[...]
    iota = lax.broadcasted_iota(jnp.int32, logits.shape, 1)
    m0 = jnp.max(logits, axis=1, keepdims=True)
    a0 = jnp.min(jnp.where(logits == m0, iota, E), axis=1, keepdims=True)
    l2 = jnp.where(iota == a0, -jnp.inf, logits)
    m1 = jnp.max(l2, axis=1, keepdims=True)
    a1 = jnp.min(jnp.where(l2 == m1, iota, E), axis=1, keepdims=True)
    e1 = jnp.exp(m1 - m0)
    s = 1.0 + e1
    w0 = 1.0 / s
    w1 = e1 / s
    oh0 = (iota == a0)
    oh1 = (iota == a1)
    scores_ref[...] = (jnp.where(oh0, w0, 0.0) + jnp.where(oh1, w1, 0.0))
    w0_ref[...] = jnp.broadcast_to(w0, (T, 16))
    w1_ref[...] = jnp.broadcast_to(w1, (T, 16))

    oh0_i = oh0.astype(jnp.int32)
    oh1_i = oh1.astype(jnp.int32)
    csum0 = _incl_scan_rows(oh0_i)
    csum1 = _incl_scan_rows(oh1_i)
    counts0 = csum0[T - 1:T, :]
    counts1 = csum1[T - 1:T, :]
    counts = counts0 + counts1
    padded = ((counts + (BLK - 1)) // BLK) * BLK
    nblk = padded // BLK
    end_blk = _incl_scan_lanes(nblk)
    offsets = (end_blk - nblk) * BLK
    na_ref[...] = end_blk[:, E - 1:E]

    rank0 = jnp.sum(oh0_i * csum0, axis=1, keepdims=True) - 1
    rank1 = jnp.sum(oh1_i * csum1, axis=1, keepdims=True) - 1
    offs0 = jnp.sum(oh0_i * offsets, axis=1, keepdims=True)
    offs1 = jnp.sum(oh1_i * offsets, axis=1, keepdims=True)
    c0sel = jnp.sum(oh1_i * counts0, axis=1, keepdims=True)
    d0_ref[...] = offs0 + rank0
    d1_ref[...] = offs1 + c0sel + rank1

    iota_b = lax.broadcasted_iota(jnp.int32, (NBMAX, E), 0)
    be = jnp.sum((end_blk <= iota_b).astype(jnp.int32), axis=1, keepdims=True)
    be_ref[...] = jnp.minimum(be, E - 1)


def _plan(x, router_w, router_b):
    return pl.pallas_call(
        _plan_kernel,
        out_shape=[
            jax.ShapeDtypeStruct((T, E), jnp.float32),
            jax.ShapeDtypeStruct((T, 1), jnp.int32),
            jax.ShapeDtypeStruct((T, 1), jnp.int32),
            jax.ShapeDtypeStruct((T, 16), jnp.float32),
            jax.ShapeDtypeStruct((T, 16), jnp.float32),
            jax.ShapeDtypeStruct((NBMAX, 1), jnp.int32),
            jax.ShapeDtypeStruct((1, 1), jnp.int32),
        ],
    )(x, router_w, router_b.reshape(1, E))


def _dispatch_body(x_hbm, d0_hbm, d1_hbm, xs_hbm,
                   xv, i0v, i1v, sem0, sem1):
    wid = lax.axis_index("s") * 2 + lax.axis_index("c")
    base = wid * TPW
    pltpu.sync_copy(x_hbm.at[pl.ds(base, TPW)], xv)
    pltpu.sync_copy(d0_hbm.at[pl.ds(base, TPW)], i0v)
    pltpu.sync_copy(d1_hbm.at[pl.ds(base, TPW)], i1v)
    c0 = pltpu.async_copy(xv, xs_hbm.at[i0v], sem0)
    c1 = pltpu.async_copy(xv, xs_hbm.at[i1v], sem1)
    c0.wait()
    c1.wait()


def _dispatch(x, d0f, d1f):
    k = functools.partial(
        pl.kernel,
        out_type=jax.ShapeDtypeStruct((PMAX, H), jnp.float32),
        mesh=plsc.VectorSubcoreMesh(core_axis_name="c", subcore_axis_name="s"),
        scratch_types=[
            pltpu.VMEM((TPW, H), jnp.float32),
            pltpu.VMEM((TPW,), jnp.int32),
            pltpu.VMEM((TPW,), jnp.int32),
            pltpu.SemaphoreType.DMA,
            pltpu.SemaphoreType.DMA,
        ],
    )(_dispatch_body)
    return k(x, d0f, d1f)


def _gmlp_kernel(be_ref, na_ref, xs_ref, gw_ref, gb_ref, uw_ref, ub_ref,
                 dw_ref, db_ref, y_ref):
    b = pl.program_id(0)

    @pl.when(b < na_ref[0])
    def _():
        x = xs_ref[...]
        gate = _bdot(x, gw_ref[0]) + gb_ref[0]
        up = _bdot(x, uw_ref[0]) + ub_ref[0]
        gate = jnp.minimum(gate, LIMIT)
        up = jnp.clip(up, -LIMIT, LIMIT)
        glu = gate * (1.0 / (1.0 + jnp.exp(-ALPHA * gate)))
        act = (up + 1.0) * glu
        y_ref[...] = _bdot(act, dw_ref[0]) + db_ref[0]


def _gmlp(be, na, xs, gate_w, gate_b, up_w, up_b, down_w, down_b):
    def _bc(b, be_r, na_r):
        return jnp.minimum(b, na_r[0] - 1)

    grid_spec = pltpu.PrefetchScalarGridSpec(
        num_scalar_prefetch=2,
        grid=(NBMAX,),
        in_specs=[
            pl.BlockSpec((BLK, H), lambda b, be_r, na_r: (_bc(b, be_r, na_r), 0)),
            pl.BlockSpec((1, I, H),
                         lambda b, be_r, na_r: (be_r[_bc(b, be_r, na_r)], 0, 0)),
            pl.BlockSpec((1, 1, I),
                         lambda b, be_r, na_r: (be_r[_bc(b, be_r, na_r)], 0, 0)),
            pl.BlockSpec((1, I, H),
                         lambda b, be_r, na_r: (be_r[_bc(b, be_r, na_r)], 0, 0)),
            pl.BlockSpec((1, 1, I),
                         lambda b, be_r, na_r: (be_r[_bc(b, be_r, na_r)], 0, 0)),
            pl.BlockSpec((1, H, I),
                         lambda b, be_r, na_r: (be_r[_bc(b, be_r, na_r)], 0, 0)),
            pl.BlockSpec((1, 1, H),
                         lambda b, be_r, na_r: (be_r[_bc(b, be_r, na_r)], 0, 0)),
        ],
        out_specs=pl.BlockSpec((BLK, H),
                               lambda b, be_r, na_r: (_bc(b, be_r, na_r), 0)),
    )
    return pl.pallas_call(
        _gmlp_kernel,
        grid_spec=grid_spec,
        out_shape=jax.ShapeDtypeStruct((PMAX, H), jnp.float32),
        compiler_params=pltpu.CompilerParams(
            dimension_semantics=("arbitrary",)),
    )(be, na, xs, gate_w, gate_b.reshape(E, 1, I), up_w,
      up_b.reshape(E, 1, I), down_w, down_b.reshape(E, 1, H))


def _combine_body(y_hbm, d0_hbm, d1_hbm, w0_hbm, w1_hbm, out_hbm,
                  g0, g1, wv0, wv1, j0, j1, sem0, sem1):
    wid = lax.axis_index("s") * 2 + lax.axis_index("c")
    CH = 32
    for c in range(TPW // CH):
        tbase = wid * TPW + c * CH
        pltpu.sync_copy(d0_hbm.at[pl.ds(tbase, CH)], j0)
        pltpu.sync_copy(d1_hbm.at[pl.ds(tbase, CH)], j1)
        pltpu.sync_copy(w0_hbm.at[pl.ds(tbase, CH)], wv0)
        pltpu.sync_copy(w1_hbm.at[pl.ds(tbase, CH)], wv1)
        c0 = pltpu.async_copy(y_hbm.at[j0], g0, sem0)
        c1 = pltpu.async_copy(y_hbm.at[j1], g1, sem1)
        c0.wait()
        c1.wait()

        def row(r, _):
            wa = wv0[r, :]
            wb = wv1[r, :]
            for v in range(H // 16):
                sl = pl.ds(v * 16, 16)
                g0[r, sl] = wa * g0[r, sl] + wb * g1[r, sl]
            return 0

        lax.fori_loop(0, CH, row, 0)
        pltpu.sync_copy(g0, out_hbm.at[pl.ds(tbase, CH)])


def _combine(y, d0f, d1f, w0r, w1r):
    k = functools.partial(
        pl.kernel,
        out_type=jax.ShapeDtypeStruct((T, H), jnp.float32),
        mesh=plsc.VectorSubcoreMesh(core_axis_name="c", subcore_axis_name="s"),
        scratch_types=[
            pltpu.VMEM((32, H), jnp.float32),
            pltpu.VMEM((32, H), jnp.float32),
            pltpu.VMEM((32, 16), jnp.float32),
            pltpu.VMEM((32, 16), jnp.float32),
            pltpu.VMEM((32,), jnp.int32),
            pltpu.VMEM((32,), jnp.int32),
            pltpu.SemaphoreType.DMA,
            pltpu.SemaphoreType.DMA,
        ],
    )(_combine_body)
    return k(y, d0f, d1f, w0r, w1r)


def _add_kernel(a_ref, b_ref, wa_ref, wb_ref, o_ref):
    o_ref[...] = wa_ref[...] * a_ref[...] + wb_ref[...] * b_ref[...]


def _add(a, b, wa, wb):
    return pl.pallas_call(
        _add_kernel,
        grid=(8,),
        in_specs=[
            pl.BlockSpec((T // 8, H), lambda i: (i, 0)),
            pl.BlockSpec((T // 8, H), lambda i: (i, 0)),
            pl.BlockSpec((T // 8, 1), lambda i: (i, 0)),
            pl.BlockSpec((T // 8, 1), lambda i: (i, 0)),
        ],
        out_specs=pl.BlockSpec((T // 8, H), lambda i: (i, 0)),
        out_shape=jax.ShapeDtypeStruct((T, H), jnp.float32),
    )(a, b, wa, wb)


@jax.jit
def kernel(hidden_states, router_w, router_b, gate_w, gate_b, up_w, up_b,
           down_w, down_b):
    B, Tq, Hq = hidden_states.shape
    x = hidden_states.reshape(T, H)

    scores, d0, d1, w0, w1, be, na = _plan(x, router_w, router_b)
    d0f = d0.reshape(T)
    d1f = d1.reshape(T)

    xs = _dispatch(x, d0f, d1f)
    y = _gmlp(be.reshape(NBMAX), na.reshape(1), xs,
              gate_w, gate_b, up_w, up_b, down_w, down_b)
    out = _combine(y, d0f, d1f, w0, w1)

    return (out.reshape(B, Tq, Hq), scores)


# speedup vs baseline: 2.2886x; 1.1082x over previous
import functools

import jax
import jax.numpy as jnp
from jax import lax
from jax.experimental import pallas as pl
from jax.experimental.pallas import tpu as pltpu
from jax.experimental.pallas import tpu_sc as plsc

TOPK = 2
ALPHA = 1.702
LIMIT = 7.0

T = 2048
H = 1024
I = 1024
E = 8
BLK = 512
A = T * TOPK
PMAX = A + E * BLK
NBMAX = A // BLK + E
NW = 32
TPW = T // NW


def _bdot(a, b):
    return jax.lax.dot_general(
        a, b, (((1,), (1,)), ((), ())),
        preferred_element_type=jnp.float32)


def _incl_scan_rows(m):
    d = 1
    n = m.shape[0]
    while d < n:
        z = jnp.zeros((d, m.shape[1]), m.dtype)
        m = m + jnp.concatenate([z, m[:-d, :]], axis=0)
        d *= 2
    return m


def _incl_scan_lanes(v):
    d = 1
    n = v.shape[1]
    while d < n:
        z = jnp.zeros((v.shape[0], d), v.dtype)
        v = v + jnp.concatenate([z, v[:, :-d]], axis=1)
        d *= 2
    return v


def _plan_kernel(x_ref, rw_ref, rb_ref, scores_ref, d0_ref, d1_ref,
                 w0_ref, w1_ref, be_ref, na_ref):
    x = x_ref[...]
    logits = _bdot(x, rw_ref[...]) + rb_ref[...]
    iota = lax.broadcasted_iota(jnp.int32, logits.shape, 1)
    m0 = jnp.max(logits, axis=1, keepdims=True)
    a0 = jnp.min(jnp.where(logits == m0, iota, E), axis=1, keepdims=True)
    l2 = jnp.where(iota == a0, -jnp.inf, logits)
    m1 = jnp.max(l2, axis=1, keepdims=True)
    a1 = jnp.min(jnp.where(l2 == m1, iota, E), axis=1, keepdims=True)
    e1 = jnp.exp(m1 - m0)
    s = 1.0 + e1
    w0 = 1.0 / s
    w1 = e1 / s
    oh0 = (iota == a0)
    oh1 = (iota == a1)
    scores_ref[...] = (jnp.where(oh0, w0, 0.0) + jnp.where(oh1, w1, 0.0))
    w0_ref[...] = jnp.broadcast_to(w0, (T, 16))
    w1_ref[...] = jnp.broadcast_to(w1, (T, 16))

    oh0_i = oh0.astype(jnp.int32)
    oh1_i = oh1.astype(jnp.int32)
    csum0 = _incl_scan_rows(oh0_i)
    csum1 = _incl_scan_rows(oh1_i)
    counts0 = csum0[T - 1:T, :]
    counts1 = csum1[T - 1:T, :]
    counts = counts0 + counts1
    padded = ((counts + (BLK - 1)) // BLK) * BLK
    nblk = padded // BLK
    end_blk = _incl_scan_lanes(nblk)
    offsets = (end_blk - nblk) * BLK
    na_ref[...] = end_blk[:, E - 1:E]

    rank0 = jnp.sum(oh0_i * csum0, axis=1, keepdims=True) - 1
    rank1 = jnp.sum(oh1_i * csum1, axis=1, keepdims=True) - 1
    offs0 = jnp.sum(oh0_i * offsets, axis=1, keepdims=True)
    offs1 = jnp.sum(oh1_i * offsets, axis=1, keepdims=True)
    c0sel = jnp.sum(oh1_i * counts0, axis=1, keepdims=True)
    d0_ref[...] = offs0 + rank0
    d1_ref[...] = offs1 + c0sel + rank1

    iota_b = lax.broadcasted_iota(jnp.int32, (NBMAX, E), 0)
    be = jnp.sum((end_blk <= iota_b).astype(jnp.int32), axis=1, keepdims=True)
    be_ref[...] = jnp.minimum(be, E - 1)


def _plan(x, router_w, router_b):
    return pl.pallas_call(
        _plan_kernel,
        out_shape=[
            jax.ShapeDtypeStruct((T, E), jnp.float32),
            jax.ShapeDtypeStruct((T, 1), jnp.int32),
            jax.ShapeDtypeStruct((T, 1), jnp.int32),
            jax.ShapeDtypeStruct((T, 16), jnp.float32),
            jax.ShapeDtypeStruct((T, 16), jnp.float32),
            jax.ShapeDtypeStruct((NBMAX, 1), jnp.int32),
            jax.ShapeDtypeStruct((1, 1), jnp.int32),
        ],
    )(x, router_w, router_b.reshape(1, E))


def _dispatch_body(x_hbm, d0_hbm, d1_hbm, xs_hbm,
                   xv, i0v, i1v, sem0, sem1):
    wid = lax.axis_index("s") * 2 + lax.axis_index("c")
    base = wid * TPW
    pltpu.sync_copy(x_hbm.at[pl.ds(base, TPW)], xv)
    pltpu.sync_copy(d0_hbm.at[pl.ds(base, TPW)], i0v)
    pltpu.sync_copy(d1_hbm.at[pl.ds(base, TPW)], i1v)
    c0 = pltpu.async_copy(xv, xs_hbm.at[i0v], sem0)
    c1 = pltpu.async_copy(xv, xs_hbm.at[i1v], sem1)
    c0.wait()
    c1.wait()


def _dispatch(x, d0f, d1f):
    k = functools.partial(
        pl.kernel,
        out_type=jax.ShapeDtypeStruct((PMAX, H), jnp.float32),
        mesh=plsc.VectorSubcoreMesh(core_axis_name="c", subcore_axis_name="s"),
        scratch_types=[
            pltpu.VMEM((TPW, H), jnp.float32),
            pltpu.VMEM((TPW,), jnp.int32),
            pltpu.VMEM((TPW,), jnp.int32),
            pltpu.SemaphoreType.DMA,
            pltpu.SemaphoreType.DMA,
        ],
    )(_dispatch_body)
    return k(x, d0f, d1f)


def _gmlp_kernel(be_ref, na_ref, xs_ref, gw_ref, gb_ref, uw_ref, ub_ref,
                 dw_ref, db_ref, y_ref):
    b = pl.program_id(0)

    @pl.when(b < na_ref[0])
    def _():
        x = xs_ref[...]
        gate = _bdot(x, gw_ref[0]) + gb_ref[0]
        up = _bdot(x, uw_ref[0]) + ub_ref[0]
        gate = jnp.minimum(gate, LIMIT)
        up = jnp.clip(up, -LIMIT, LIMIT)
        glu = gate * (1.0 / (1.0 + jnp.exp(-ALPHA * gate)))
        act = (up + 1.0) * glu
        y_ref[...] = _bdot(act, dw_ref[0]) + db_ref[0]


def _gmlp(be, na, xs, gate_w, gate_b, up_w, up_b, down_w, down_b):
    def _bc(b, be_r, na_r):
        return jnp.minimum(b, na_r[0] - 1)

    grid_spec = pltpu.PrefetchScalarGridSpec(
        num_scalar_prefetch=2,
        grid=(NBMAX,),
        in_specs=[
            pl.BlockSpec((BLK, H), lambda b, be_r, na_r: (_bc(b, be_r, na_r), 0)),
            pl.BlockSpec((1, I, H),
                         lambda b, be_r, na_r: (be_r[_bc(b, be_r, na_r)], 0, 0)),
            pl.BlockSpec((1, 1, I),
                         lambda b, be_r, na_r: (be_r[_bc(b, be_r, na_r)], 0, 0)),
            pl.BlockSpec((1, I, H),
                         lambda b, be_r, na_r: (be_r[_bc(b, be_r, na_r)], 0, 0)),
            pl.BlockSpec((1, 1, I),
                         lambda b, be_r, na_r: (be_r[_bc(b, be_r, na_r)], 0, 0)),
            pl.BlockSpec((1, H, I),
                         lambda b, be_r, na_r: (be_r[_bc(b, be_r, na_r)], 0, 0)),
            pl.BlockSpec((1, 1, H),
                         lambda b, be_r, na_r: (be_r[_bc(b, be_r, na_r)], 0, 0)),
        ],
        out_specs=pl.BlockSpec((BLK, H),
                               lambda b, be_r, na_r: (_bc(b, be_r, na_r), 0)),
    )
    return pl.pallas_call(
        _gmlp_kernel,
        grid_spec=grid_spec,
        out_shape=jax.ShapeDtypeStruct((PMAX, H), jnp.float32),
        compiler_params=pltpu.CompilerParams(
            dimension_semantics=("arbitrary",)),
    )(be, na, xs, gate_w, gate_b.reshape(E, 1, I), up_w,
      up_b.reshape(E, 1, I), down_w, down_b.reshape(E, 1, H))


def _combine_body(y_hbm, d0_hbm, d1_hbm, w0_hbm, w1_hbm, out_hbm,
                  g0, g1, wv0, wv1, j0, j1, sem0, sem1):
    wid = lax.axis_index("s") * 2 + lax.axis_index("c")
    CH = 32
    for c in range(TPW // CH):
        tbase = wid * TPW + c * CH
        pltpu.sync_copy(d0_hbm.at[pl.ds(tbase, CH)], j0)
        pltpu.sync_copy(d1_hbm.at[pl.ds(tbase, CH)], j1)
        pltpu.sync_copy(w0_hbm.at[pl.ds(tbase, CH)], wv0)
        pltpu.sync_copy(w1_hbm.at[pl.ds(tbase, CH)], wv1)
        c0 = pltpu.async_copy(y_hbm.at[j0], g0, sem0)
        c1 = pltpu.async_copy(y_hbm.at[j1], g1, sem1)
        c0.wait()
        c1.wait()

        def row(r, _):
            wa = wv0[r, :]
            wb = wv1[r, :]
            for v in range(H // 16):
                sl = pl.ds(v * 16, 16)
                g0[r, sl] = wa * g0[r, sl] + wb * g1[r, sl]
            return 0

        lax.fori_loop(0, CH, row, 0)
        pltpu.sync_copy(g0, out_hbm.at[pl.ds(tbase, CH)])


def _combine(y, d0f, d1f, w0r, w1r):
    k = functools.partial(
        pl.kernel,
        out_type=jax.ShapeDtypeStruct((T, H), jnp.float32),
        mesh=plsc.VectorSubcoreMesh(core_axis_name="c", subcore_axis_name="s"),
        scratch_types=[
            pltpu.VMEM((32, H), jnp.float32),
            pltpu.VMEM((32, H), jnp.float32),
            pltpu.VMEM((32, 16), jnp.float32),
            pltpu.VMEM((32, 16), jnp.float32),
            pltpu.VMEM((32,), jnp.int32),
            pltpu.VMEM((32,), jnp.int32),
            pltpu.SemaphoreType.DMA,
            pltpu.SemaphoreType.DMA,
        ],
    )(_combine_body)
    return k(y, d0f, d1f, w0r, w1r)


def _add_kernel(a_ref, b_ref, wa_ref, wb_ref, o_ref):
    o_ref[...] = wa_ref[...] * a_ref[...] + wb_ref[...] * b_ref[...]


def _add(a, b, wa, wb):
    return pl.pallas_call(
        _add_kernel,
        grid=(8,),
        in_specs=[
            pl.BlockSpec((T // 8, H), lambda i: (i, 0)),
            pl.BlockSpec((T // 8, H), lambda i: (i, 0)),
            pl.BlockSpec((T // 8, 1), lambda i: (i, 0)),
            pl.BlockSpec((T // 8, 1), lambda i: (i, 0)),
        ],
        out_specs=pl.BlockSpec((T // 8, H), lambda i: (i, 0)),
        out_shape=jax.ShapeDtypeStruct((T, H), jnp.float32),
    )(a, b, wa, wb)


@jax.jit
def kernel(hidden_states, router_w, router_b, gate_w, gate_b, up_w, up_b,
           down_w, down_b):
    B, Tq, Hq = hidden_states.shape
    x = hidden_states.reshape(T, H)

    scores, d0, d1, w0, w1, be, na = _plan(x, router_w, router_b)
    d0f = d0.reshape(T)
    d1f = d1.reshape(T)

    xs = _dispatch(x, d0f, d1f)
    y = _gmlp(be.reshape(NBMAX), na.reshape(1), xs,
              gate_w, gate_b, up_w, up_b, down_w, down_b)
    out = _combine(y, d0f, d1f, w0, w1)

    return (out.reshape(B, Tq, Hq), scores)
